# Initial kernel scaffold; baseline (speedup 1.0000x reference)
import jax
import jax.numpy as jnp
from jax.experimental import pallas as pl


def kernel(x, edge_index, batch, W1, b1, g1, be1, rm1, rv1, W2, b2, g2, be2, rm2, rv2, W3, b3, g3, be3, rm3, rv3, mW1, mb1, mW2, mb2):
    raise NotImplementedError("write your pallas kernel here")


# speedup vs baseline: 9.7975x; 9.7975x over previous
import functools

import jax
import jax.numpy as jnp
from jax import lax
from jax.experimental import pallas as pl
from jax.experimental.pallas import tpu as pltpu
from jax.experimental.pallas import tpu_sc as plsc

N = 10000
E = 320000
F_IN = 128
H = 256
HH = 128
G = 64
EPS = 1e-5
NP = 10240
NSUB = 16
W_E = 128

EPS_DEG = E // (2 * NSUB)
DEG_FULL = EPS_DEG // W_E
DEG_TAIL = EPS_DEG - DEG_FULL * W_E
EPS_AGG = E // NSUB
AGG_FULL = EPS_AGG // W_E
AGG_TAIL = EPS_AGG - AGG_FULL * W_E
ROWS_A = 640
ROWS_B = N - 15 * ROWS_A

NB = 2000
NGRID = N // NB

_mesh = plsc.VectorSubcoreMesh(core_axis_name="c", subcore_axis_name="s",
                               num_cores=2, num_subcores=NSUB)


def _deg_body(col_hbm, deg0_hbm, deg1_hbm, zbuf, ones_v, cidx, ones_t, cidx_t,
              deg_sp):
    c = lax.axis_index("c")
    s = lax.axis_index("s")

    def fz(i, _):
        zbuf[pl.ds(i * 16, 16)] = jnp.zeros((16,), jnp.float32)
        return 0
    lax.fori_loop(0, 40, fz, 0)

    def fo(i, _):
        ones_v[pl.ds(i * 16, 16)] = jnp.ones((16,), jnp.float32)
        return 0
    lax.fori_loop(0, W_E // 16, fo, 0)
    ones_t[...] = jnp.ones((DEG_TAIL,), jnp.float32)

    pltpu.sync_copy(zbuf, deg_sp.at[pl.ds(pl.multiple_of(s * 640, 8), 640)])
    plsc.subcore_barrier()

    base = (c * NSUB + s) * EPS_DEG

    def win(w, _):
        off = pl.multiple_of(base + w * W_E, 8)
        pltpu.sync_copy(col_hbm.at[pl.ds(off, W_E)], cidx)
        pltpu.sync_copy(ones_v, deg_sp.at[cidx], add=True)
        return 0
    lax.fori_loop(0, DEG_FULL, win, 0)
    off = pl.multiple_of(base + DEG_FULL * W_E, 8)
    pltpu.sync_copy(col_hbm.at[pl.ds(off, DEG_TAIL)], cidx_t)
    pltpu.sync_copy(ones_t, deg_sp.at[cidx_t], add=True)

    plsc.subcore_barrier()

    @pl.when((s == 0) & (c == 0))
    def _():
        pltpu.sync_copy(deg_sp, deg0_hbm)

    @pl.when((s == 0) & (c == 1))
    def _():
        pltpu.sync_copy(deg_sp, deg1_hbm)


_deg_call = pl.kernel(
    _deg_body,
    out_type=(
        jax.ShapeDtypeStruct((NP,), jnp.float32),
        jax.ShapeDtypeStruct((NP,), jnp.float32),
    ),
    mesh=_mesh,
    scratch_types=[
        pltpu.VMEM((640,), jnp.float32),
        pltpu.VMEM((W_E,), jnp.float32),
        pltpu.VMEM((W_E,), jnp.int32),
        pltpu.VMEM((DEG_TAIL,), jnp.float32),
        pltpu.VMEM((DEG_TAIL,), jnp.int32),
        pltpu.VMEM_SHARED((NP,), jnp.float32),
    ],
)


def _copy_rows(src, dst, s):
    r0 = pl.multiple_of(s * ROWS_A, 8)

    @pl.when(s < 15)
    def _():
        pltpu.sync_copy(src.at[pl.ds(r0, ROWS_A)], dst.at[pl.ds(r0, ROWS_A)])

    @pl.when(s == 15)
    def _():
        pltpu.sync_copy(src.at[pl.ds(15 * ROWS_A, ROWS_B)],
                        dst.at[pl.ds(15 * ROWS_A, ROWS_B)])


def _agg_body(u0, u1, row_hbm, col_hbm, agg0, agg1,
              ridx, cidx, ridx_t, cidx_t, rows, rows_t, agg_sp, sem):
    c = lax.axis_index("c")
    s = lax.axis_index("s")

    @pl.when(c == 0)
    def _():
        _copy_rows(u0, agg_sp, s)

    @pl.when(c == 1)
    def _():
        _copy_rows(u1, agg_sp, s)

    plsc.subcore_barrier()

    ebase = s * EPS_AGG

    def edge_loop(u_ref):
        def win(w, _):
            off = pl.multiple_of(ebase + w * W_E, 8)
            pltpu.sync_copy(row_hbm.at[pl.ds(off, W_E)], ridx)
            pltpu.sync_copy(col_hbm.at[pl.ds(off, W_E)], cidx)
            pltpu.async_copy(u_ref.at[ridx], rows, sem).wait()
            pltpu.sync_copy(rows, agg_sp.at[cidx], add=True)
            return 0
        lax.fori_loop(0, AGG_FULL, win, 0)
        off = pl.multiple_of(ebase + AGG_FULL * W_E, 8)
        pltpu.sync_copy(row_hbm.at[pl.ds(off, AGG_TAIL)], ridx_t)
        pltpu.sync_copy(col_hbm.at[pl.ds(off, AGG_TAIL)], cidx_t)
        pltpu.async_copy(u_ref.at[ridx_t], rows_t, sem).wait()
        pltpu.sync_copy(rows_t, agg_sp.at[cidx_t], add=True)

    @pl.when(c == 0)
    def _():
        edge_loop(u0)

    @pl.when(c == 1)
    def _():
        edge_loop(u1)

    plsc.subcore_barrier()

    @pl.when(c == 0)
    def _():
        _copy_rows(agg_sp, agg0, s)

    @pl.when(c == 1)
    def _():
        _copy_rows(agg_sp, agg1, s)


_agg_call = pl.kernel(
    _agg_body,
    out_type=(
        jax.ShapeDtypeStruct((N, HH), jnp.float32),
        jax.ShapeDtypeStruct((N, HH), jnp.float32),
    ),
    mesh=_mesh,
    scratch_types=[
        pltpu.VMEM((W_E,), jnp.int32),
        pltpu.VMEM((W_E,), jnp.int32),
        pltpu.VMEM((AGG_TAIL,), jnp.int32),
        pltpu.VMEM((AGG_TAIL,), jnp.int32),
        pltpu.VMEM((W_E, HH), jnp.float32),
        pltpu.VMEM((AGG_TAIL, HH), jnp.float32),
        pltpu.VMEM_SHARED((N, HH), jnp.float32),
        pltpu.SemaphoreType.DMA,
    ],
)


def _prep_body(deg0_ref, deg1_ref, dinv_ref):
    d = deg0_ref[...] + deg1_ref[...] + 1.0
    r = lax.rsqrt(d)
    r = r * (1.5 - 0.5 * d * r * r)
    dinv_ref[...] = r[:, None]


_prep_call = pl.pallas_call(
    _prep_body,
    out_shape=jax.ShapeDtypeStruct((NP, 1), jnp.float32),
    in_specs=[
        pl.BlockSpec((NP,), lambda: (0,)),
        pl.BlockSpec((NP,), lambda: (0,)),
    ],
    out_specs=pl.BlockSpec((NP, 1), lambda: (0, 0)),
)


def _mm1_body(x_ref, w_ref, dinv_ref, u0_ref, u1_ref):
    h = jnp.dot(x_ref[...], w_ref[...], preferred_element_type=jnp.float32,
                precision=lax.Precision.DEFAULT)
    u = h * dinv_ref[...]
    u0_ref[...] = u[:, :HH]
    u1_ref[...] = u[:, HH:]


_mm1_call = pl.pallas_call(
    _mm1_body,
    grid=(NGRID,),
    out_shape=(
        jax.ShapeDtypeStruct((N, HH), jnp.float32),
        jax.ShapeDtypeStruct((N, HH), jnp.float32),
    ),
    in_specs=[
        pl.BlockSpec((NB, F_IN), lambda i: (i, 0)),
        pl.BlockSpec((F_IN, H), lambda i: (0, 0)),
        pl.BlockSpec((NB, 1), lambda i: (i, 0)),
    ],
    out_specs=(
        pl.BlockSpec((NB, HH), lambda i: (i, 0)),
        pl.BlockSpec((NB, HH), lambda i: (i, 0)),
    ),
)


def _mid_body(a0_ref, a1_ref, dinv_ref, vs_ref, vo_ref, w_ref,
              n0_ref, n1_ref):
    a = jnp.concatenate([a0_ref[...], a1_ref[...]], axis=1)
    h = jnp.maximum(a * dinv_ref[...] * vs_ref[...] + vo_ref[...], 0.0)
    nu = jnp.dot(h, w_ref[...], preferred_element_type=jnp.float32,
                precision=lax.Precision.DEFAULT) * dinv_ref[...]
    n0_ref[...] = nu[:, :HH]
    n1_ref[...] = nu[:, HH:]


_mid_call = pl.pallas_call(
    _mid_body,
    grid=(NGRID,),
    out_shape=(
        jax.ShapeDtypeStruct((N, HH), jnp.float32),
        jax.ShapeDtypeStruct((N, HH), jnp.float32),
    ),
    in_specs=[
        pl.BlockSpec((NB, HH), lambda i: (i, 0)),
        pl.BlockSpec((NB, HH), lambda i: (i, 0)),
        pl.BlockSpec((NB, 1), lambda i: (i, 0)),
        pl.BlockSpec((1, H), lambda i: (0, 0)),
        pl.BlockSpec((1, H), lambda i: (0, 0)),
        pl.BlockSpec((H, H), lambda i: (0, 0)),
    ],
    out_specs=(
        pl.BlockSpec((NB, HH), lambda i: (i, 0)),
        pl.BlockSpec((NB, HH), lambda i: (i, 0)),
    ),
)


def _pool_body(a0_ref, a1_ref, dinv_ref, vs_ref, vo_ref,
               batch_ref, mw1_ref, mb1_ref, mw2_ref, mb2_ref,
               out_ref, s_acc, c_acc):
    i = pl.program_id(0)

    @pl.when(i == 0)
    def _():
        s_acc[...] = jnp.zeros_like(s_acc)
        c_acc[...] = jnp.zeros_like(c_acc)

    a = jnp.concatenate([a0_ref[...], a1_ref[...]], axis=1)
    h = jnp.maximum(a * dinv_ref[...] * vs_ref[...] + vo_ref[...], 0.0)
    b = batch_ref[...][:, 0]
    p = (b[None, :] == lax.broadcasted_iota(jnp.int32, (G, NB), 0)
         ).astype(jnp.float32)
    s_acc[...] += jnp.dot(p, h, preferred_element_type=jnp.float32,
                precision=lax.Precision.HIGHEST)
    c_acc[...] += jnp.sum(p, axis=1, keepdims=True)

    @pl.when(i == NGRID - 1)
    def _():
        pooled = s_acc[...] / jnp.maximum(c_acc[...], 1.0)
        z = jnp.maximum(
            jnp.dot(pooled, mw1_ref[...], preferred_element_type=jnp.float32,
                precision=lax.Precision.DEFAULT)
            + mb1_ref[...], 0.0)
        out_ref[...] = (
            jnp.dot(z, mw2_ref[...], preferred_element_type=jnp.float32,
                precision=lax.Precision.DEFAULT)
            + mb2_ref[...])


_pool_call = pl.pallas_call(
    _pool_body,
    grid=(NGRID,),
    out_shape=jax.ShapeDtypeStruct((G, 1), jnp.float32),
    in_specs=[
        pl.BlockSpec((NB, HH), lambda i: (i, 0)),
        pl.BlockSpec((NB, HH), lambda i: (i, 0)),
        pl.BlockSpec((NB, 1), lambda i: (i, 0)),
        pl.BlockSpec((1, H), lambda i: (0, 0)),
        pl.BlockSpec((1, H), lambda i: (0, 0)),
        pl.BlockSpec((NB, 1), lambda i: (i, 0)),
        pl.BlockSpec((H, H), lambda i: (0, 0)),
        pl.BlockSpec((1, H), lambda i: (0, 0)),
        pl.BlockSpec((H, 1), lambda i: (0, 0)),
        pl.BlockSpec((1, 1), lambda i: (0, 0)),
    ],
    out_specs=pl.BlockSpec((G, 1), lambda i: (0, 0)),
    scratch_shapes=[
        pltpu.VMEM((G, H), jnp.float32),
        pltpu.VMEM((G, 1), jnp.float32),
    ],
)


def kernel(x, edge_index, batch, W1, b1, g1, be1, rm1, rv1,
           W2, b2, g2, be2, rm2, rv2, W3, b3, g3, be3, rm3, rv3,
           mW1, mb1, mW2, mb2):
    row = edge_index[0].astype(jnp.int32)
    col = edge_index[1].astype(jnp.int32)
    batch2 = batch.astype(jnp.int32).reshape(N, 1)

    def fold(gv, bev, rmv, rvv, bv):
        sc = (gv * lax.rsqrt(rvv + EPS)).reshape(1, H)
        off = (bv * sc.reshape(H) + bev - rmv * sc.reshape(H)).reshape(1, H)
        return sc, off

    sc1, vo1 = fold(g1, be1, rm1, rv1, b1)
    sc2, vo2 = fold(g2, be2, rm2, rv2, b2)
    sc3, vo3 = fold(g3, be3, rm3, rv3, b3)

    deg0, deg1 = _deg_call(col)
    dinv = _prep_call(deg0, deg1)

    u0, u1 = _mm1_call(x, W1, dinv)
    a0, a1 = _agg_call(u0, u1, row, col)
    u0, u1 = _mid_call(a0, a1, dinv, sc1, vo1, W2)
    a0, a1 = _agg_call(u0, u1, row, col)
    u0, u1 = _mid_call(a0, a1, dinv, sc2, vo2, W3)
    a0, a1 = _agg_call(u0, u1, row, col)
    logit = _pool_call(a0, a1, dinv, sc3, vo3, batch2,
                       mW1, mb1.reshape(1, H), mW2, mb2.reshape(1, 1))
    return logit.reshape(G)


# speedup vs baseline: 11.9168x; 1.2163x over previous
import functools

import jax
import jax.numpy as jnp
from jax import lax
from jax.experimental import pallas as pl
from jax.experimental.pallas import tpu as pltpu
from jax.experimental.pallas import tpu_sc as plsc

N = 10000
E = 320000
F_IN = 128
H = 256
HH = 128
G = 64
EPS = 1e-5
NP = 10240
NSUB = 16
W_E = 128

EPS_DEG = E // (2 * NSUB)
DEG_FULL = EPS_DEG // W_E
DEG_TAIL = EPS_DEG - DEG_FULL * W_E
EPS_AGG = E // NSUB
AGG_FULL = EPS_AGG // W_E
AGG_TAIL = EPS_AGG - AGG_FULL * W_E
ROWS_A = 640
ROWS_B = N - 15 * ROWS_A

NB = 2000
NGRID = N // NB

_mesh = plsc.VectorSubcoreMesh(core_axis_name="c", subcore_axis_name="s",
                               num_cores=2, num_subcores=NSUB)


def _deg_body(col_hbm, deg0_hbm, deg1_hbm, zbuf, ones_v, cidx, ones_t, cidx_t,
              deg_sp):
    c = lax.axis_index("c")
    s = lax.axis_index("s")

    def fz(i, _):
        zbuf[pl.ds(i * 16, 16)] = jnp.zeros((16,), jnp.float32)
        return 0
    lax.fori_loop(0, 40, fz, 0)

    def fo(i, _):
        ones_v[pl.ds(i * 16, 16)] = jnp.ones((16,), jnp.float32)
        return 0
    lax.fori_loop(0, W_E // 16, fo, 0)
    ones_t[...] = jnp.ones((DEG_TAIL,), jnp.float32)

    pltpu.sync_copy(zbuf, deg_sp.at[pl.ds(pl.multiple_of(s * 640, 8), 640)])
    plsc.subcore_barrier()

    base = (c * NSUB + s) * EPS_DEG

    def win(w, _):
        off = pl.multiple_of(base + w * W_E, 8)
        pltpu.sync_copy(col_hbm.at[pl.ds(off, W_E)], cidx)
        pltpu.sync_copy(ones_v, deg_sp.at[cidx], add=True)
        return 0
    lax.fori_loop(0, DEG_FULL, win, 0)
    off = pl.multiple_of(base + DEG_FULL * W_E, 8)
    pltpu.sync_copy(col_hbm.at[pl.ds(off, DEG_TAIL)], cidx_t)
    pltpu.sync_copy(ones_t, deg_sp.at[cidx_t], add=True)

    plsc.subcore_barrier()

    @pl.when((s == 0) & (c == 0))
    def _():
        pltpu.sync_copy(deg_sp, deg0_hbm)

    @pl.when((s == 0) & (c == 1))
    def _():
        pltpu.sync_copy(deg_sp, deg1_hbm)


_deg_call = pl.kernel(
    _deg_body,
    out_type=(
        jax.ShapeDtypeStruct((NP,), jnp.float32),
        jax.ShapeDtypeStruct((NP,), jnp.float32),
    ),
    mesh=_mesh,
    scratch_types=[
        pltpu.VMEM((640,), jnp.float32),
        pltpu.VMEM((W_E,), jnp.float32),
        pltpu.VMEM((W_E,), jnp.int32),
        pltpu.VMEM((DEG_TAIL,), jnp.float32),
        pltpu.VMEM((DEG_TAIL,), jnp.int32),
        pltpu.VMEM_SHARED((NP,), jnp.float32),
    ],
)


def _copy_rows(src, dst, s):
    r0 = pl.multiple_of(s * ROWS_A, 8)

    @pl.when(s < 15)
    def _():
        pltpu.sync_copy(src.at[pl.ds(r0, ROWS_A)], dst.at[pl.ds(r0, ROWS_A)])

    @pl.when(s == 15)
    def _():
        pltpu.sync_copy(src.at[pl.ds(15 * ROWS_A, ROWS_B)],
                        dst.at[pl.ds(15 * ROWS_A, ROWS_B)])


def _agg_body(u0, u1, row_hbm, col_hbm, agg0, agg1,
              ridx, cidx, ridx_t, cidx_t, rows, rows_t, agg_sp, gsem, ssem):
    c = lax.axis_index("c")
    s = lax.axis_index("s")

    @pl.when(c == 0)
    def _():
        _copy_rows(u0, agg_sp, s)

    @pl.when(c == 1)
    def _():
        _copy_rows(u1, agg_sp, s)

    plsc.subcore_barrier()

    ebase = s * EPS_AGG

    def edge_loop(u_ref):
        def load_and_gather(w, b):
            off = pl.multiple_of(ebase + w * W_E, 8)
            pltpu.sync_copy(row_hbm.at[pl.ds(off, W_E)], ridx.at[b])
            pltpu.sync_copy(col_hbm.at[pl.ds(off, W_E)], cidx.at[b])
            pltpu.async_copy(u_ref.at[ridx.at[b]], rows.at[b], gsem)

        load_and_gather(0, 0)

        def win(w, _):
            b = lax.rem(w, 2)
            nb = 1 - b
            pltpu.make_async_copy(u_ref.at[ridx.at[b]], rows.at[b], gsem).wait()
            pltpu.async_copy(rows.at[b], agg_sp.at[cidx.at[b]], ssem, add=True)

            @pl.when(w + 1 < AGG_FULL)
            def _():
                @pl.when(w >= 1)
                def _():
                    pltpu.make_async_copy(
                        rows.at[nb], agg_sp.at[cidx.at[nb]], ssem).wait()
                load_and_gather(w + 1, nb)
            return 0
        lax.fori_loop(0, AGG_FULL, win, 0)
        lb = (AGG_FULL - 1) % 2
        pltpu.make_async_copy(rows.at[1 - lb], agg_sp.at[cidx.at[1 - lb]],
                              ssem).wait()
        pltpu.make_async_copy(rows.at[lb], agg_sp.at[cidx.at[lb]], ssem).wait()

        off = pl.multiple_of(ebase + AGG_FULL * W_E, 8)
        pltpu.sync_copy(row_hbm.at[pl.ds(off, AGG_TAIL)], ridx_t)
        pltpu.sync_copy(col_hbm.at[pl.ds(off, AGG_TAIL)], cidx_t)
        pltpu.async_copy(u_ref.at[ridx_t], rows_t, gsem).wait()
        pltpu.sync_copy(rows_t, agg_sp.at[cidx_t], add=True)

    @pl.when(c == 0)
    def _():
        edge_loop(u0)

    @pl.when(c == 1)
    def _():
        edge_loop(u1)

    plsc.subcore_barrier()

    @pl.when(c == 0)
    def _():
        _copy_rows(agg_sp, agg0, s)

    @pl.when(c == 1)
    def _():
        _copy_rows(agg_sp, agg1, s)


_agg_call = pl.kernel(
    _agg_body,
    out_type=(
        jax.ShapeDtypeStruct((N, HH), jnp.float32),
        jax.ShapeDtypeStruct((N, HH), jnp.float32),
    ),
    mesh=_mesh,
    scratch_types=[
        pltpu.VMEM((2, W_E), jnp.int32),
        pltpu.VMEM((2, W_E), jnp.int32),
        pltpu.VMEM((AGG_TAIL,), jnp.int32),
        pltpu.VMEM((AGG_TAIL,), jnp.int32),
        pltpu.VMEM((2, W_E, HH), jnp.float32),
        pltpu.VMEM((AGG_TAIL, HH), jnp.float32),
        pltpu.VMEM_SHARED((N, HH), jnp.float32),
        pltpu.SemaphoreType.DMA,
        pltpu.SemaphoreType.DMA,
    ],
)


def _prep_body(deg0_ref, deg1_ref, dinv_ref):
    d = deg0_ref[...] + deg1_ref[...] + 1.0
    r = lax.rsqrt(d)
    r = r * (1.5 - 0.5 * d * r * r)
    dinv_ref[...] = r[:, None]


_prep_call = pl.pallas_call(
    _prep_body,
    out_shape=jax.ShapeDtypeStruct((NP, 1), jnp.float32),
    in_specs=[
        pl.BlockSpec((NP,), lambda: (0,)),
        pl.BlockSpec((NP,), lambda: (0,)),
    ],
    out_specs=pl.BlockSpec((NP, 1), lambda: (0, 0)),
)


def _mm1_body(x_ref, w_ref, dinv_ref, u0_ref, u1_ref):
    h = jnp.dot(x_ref[...], w_ref[...], preferred_element_type=jnp.float32,
                precision=lax.Precision.DEFAULT)
    u = h * dinv_ref[...]
    u0_ref[...] = u[:, :HH]
    u1_ref[...] = u[:, HH:]


_mm1_call = pl.pallas_call(
    _mm1_body,
    grid=(NGRID,),
    out_shape=(
        jax.ShapeDtypeStruct((N, HH), jnp.float32),
        jax.ShapeDtypeStruct((N, HH), jnp.float32),
    ),
    in_specs=[
        pl.BlockSpec((NB, F_IN), lambda i: (i, 0)),
        pl.BlockSpec((F_IN, H), lambda i: (0, 0)),
        pl.BlockSpec((NB, 1), lambda i: (i, 0)),
    ],
    out_specs=(
        pl.BlockSpec((NB, HH), lambda i: (i, 0)),
        pl.BlockSpec((NB, HH), lambda i: (i, 0)),
    ),
)


def _mid_body(a0_ref, a1_ref, dinv_ref, vs_ref, vo_ref, w_ref,
              n0_ref, n1_ref):
    a = jnp.concatenate([a0_ref[...], a1_ref[...]], axis=1)
    h = jnp.maximum(a * dinv_ref[...] * vs_ref[...] + vo_ref[...], 0.0)
    nu = jnp.dot(h, w_ref[...], preferred_element_type=jnp.float32,
                precision=lax.Precision.DEFAULT) * dinv_ref[...]
    n0_ref[...] = nu[:, :HH]
    n1_ref[...] = nu[:, HH:]


_mid_call = pl.pallas_call(
    _mid_body,
    grid=(NGRID,),
    out_shape=(
        jax.ShapeDtypeStruct((N, HH), jnp.float32),
        jax.ShapeDtypeStruct((N, HH), jnp.float32),
    ),
    in_specs=[
        pl.BlockSpec((NB, HH), lambda i: (i, 0)),
        pl.BlockSpec((NB, HH), lambda i: (i, 0)),
        pl.BlockSpec((NB, 1), lambda i: (i, 0)),
        pl.BlockSpec((1, H), lambda i: (0, 0)),
        pl.BlockSpec((1, H), lambda i: (0, 0)),
        pl.BlockSpec((H, H), lambda i: (0, 0)),
    ],
    out_specs=(
        pl.BlockSpec((NB, HH), lambda i: (i, 0)),
        pl.BlockSpec((NB, HH), lambda i: (i, 0)),
    ),
)


def _pool_body(a0_ref, a1_ref, dinv_ref, vs_ref, vo_ref,
               batch_ref, mw1_ref, mb1_ref, mw2_ref, mb2_ref,
               out_ref, s_acc, c_acc):
    i = pl.program_id(0)

    @pl.when(i == 0)
    def _():
        s_acc[...] = jnp.zeros_like(s_acc)
        c_acc[...] = jnp.zeros_like(c_acc)

    a = jnp.concatenate([a0_ref[...], a1_ref[...]], axis=1)
    h = jnp.maximum(a * dinv_ref[...] * vs_ref[...] + vo_ref[...], 0.0)
    b = batch_ref[...][:, 0]
    p = (b[None, :] == lax.broadcasted_iota(jnp.int32, (G, NB), 0)
         ).astype(jnp.float32)
    s_acc[...] += jnp.dot(p, h, preferred_element_type=jnp.float32,
                precision=lax.Precision.HIGHEST)
    c_acc[...] += jnp.sum(p, axis=1, keepdims=True)

    @pl.when(i == NGRID - 1)
    def _():
        pooled = s_acc[...] / jnp.maximum(c_acc[...], 1.0)
        z = jnp.maximum(
            jnp.dot(pooled, mw1_ref[...], preferred_element_type=jnp.float32,
                precision=lax.Precision.DEFAULT)
            + mb1_ref[...], 0.0)
        out_ref[...] = (
            jnp.dot(z, mw2_ref[...], preferred_element_type=jnp.float32,
                precision=lax.Precision.DEFAULT)
            + mb2_ref[...])


_pool_call = pl.pallas_call(
    _pool_body,
    grid=(NGRID,),
    out_shape=jax.ShapeDtypeStruct((G, 1), jnp.float32),
    in_specs=[
        pl.BlockSpec((NB, HH), lambda i: (i, 0)),
        pl.BlockSpec((NB, HH), lambda i: (i, 0)),
        pl.BlockSpec((NB, 1), lambda i: (i, 0)),
        pl.BlockSpec((1, H), lambda i: (0, 0)),
        pl.BlockSpec((1, H), lambda i: (0, 0)),
        pl.BlockSpec((NB, 1), lambda i: (i, 0)),
        pl.BlockSpec((H, H), lambda i: (0, 0)),
        pl.BlockSpec((1, H), lambda i: (0, 0)),
        pl.BlockSpec((H, 1), lambda i: (0, 0)),
        pl.BlockSpec((1, 1), lambda i: (0, 0)),
    ],
    out_specs=pl.BlockSpec((G, 1), lambda i: (0, 0)),
    scratch_shapes=[
        pltpu.VMEM((G, H), jnp.float32),
        pltpu.VMEM((G, 1), jnp.float32),
    ],
)


def kernel(x, edge_index, batch, W1, b1, g1, be1, rm1, rv1,
           W2, b2, g2, be2, rm2, rv2, W3, b3, g3, be3, rm3, rv3,
           mW1, mb1, mW2, mb2):
    row = edge_index[0].astype(jnp.int32)
    col = edge_index[1].astype(jnp.int32)
    batch2 = batch.astype(jnp.int32).reshape(N, 1)

    def fold(gv, bev, rmv, rvv, bv):
        sc = (gv * lax.rsqrt(rvv + EPS)).reshape(1, H)
        off = (bv * sc.reshape(H) + bev - rmv * sc.reshape(H)).reshape(1, H)
        return sc, off

    sc1, vo1 = fold(g1, be1, rm1, rv1, b1)
    sc2, vo2 = fold(g2, be2, rm2, rv2, b2)
    sc3, vo3 = fold(g3, be3, rm3, rv3, b3)

    deg0, deg1 = _deg_call(col)
    dinv = _prep_call(deg0, deg1)

    u0, u1 = _mm1_call(x, W1, dinv)
    a0, a1 = _agg_call(u0, u1, row, col)
    u0, u1 = _mid_call(a0, a1, dinv, sc1, vo1, W2)
    a0, a1 = _agg_call(u0, u1, row, col)
    u0, u1 = _mid_call(a0, a1, dinv, sc2, vo2, W3)
    a0, a1 = _agg_call(u0, u1, row, col)
    logit = _pool_call(a0, a1, dinv, sc3, vo3, batch2,
                       mW1, mb1.reshape(1, H), mW2, mb2.reshape(1, 1))
    return logit.reshape(G)


# speedup vs baseline: 16.7216x; 1.4032x over previous
import functools

import jax
import jax.numpy as jnp
from jax import lax
from jax.experimental import pallas as pl
from jax.experimental.pallas import tpu as pltpu
from jax.experimental.pallas import tpu_sc as plsc

N = 10000
E = 320000
F_IN = 128
H = 256
HH = 128
G = 64
EPS = 1e-5
NP = 10240
NSUB = 16
W_E = 128

EPS_DEG = E // (2 * NSUB)
DEG_FULL = EPS_DEG // W_E
DEG_TAIL = EPS_DEG - DEG_FULL * W_E
W_A = 125
EROWS = E // W_A
SROWS = EROWS // NSUB
C_A = 40
NCHUNK = SROWS // C_A
ROWS_A = 640
ROWS_B = N - 15 * ROWS_A

NB = 2000
NGRID = N // NB

_mesh = plsc.VectorSubcoreMesh(core_axis_name="c", subcore_axis_name="s",
                               num_cores=2, num_subcores=NSUB)


def _deg_body(col_hbm, deg0_hbm, deg1_hbm, zbuf, ones_v, cidx, ones_t, cidx_t,
              deg_sp):
    c = lax.axis_index("c")
    s = lax.axis_index("s")

    def fz(i, _):
        zbuf[pl.ds(i * 16, 16)] = jnp.zeros((16,), jnp.float32)
        return 0
    lax.fori_loop(0, 40, fz, 0)

    def fo(i, _):
        ones_v[pl.ds(i * 16, 16)] = jnp.ones((16,), jnp.float32)
        return 0
    lax.fori_loop(0, W_E // 16, fo, 0)
    ones_t[...] = jnp.ones((DEG_TAIL,), jnp.float32)

    pltpu.sync_copy(zbuf, deg_sp.at[pl.ds(pl.multiple_of(s * 640, 8), 640)])
    plsc.subcore_barrier()

    base = (c * NSUB + s) * EPS_DEG

    def win(w, _):
        off = pl.multiple_of(base + w * W_E, 8)
        pltpu.sync_copy(col_hbm.at[pl.ds(off, W_E)], cidx)
        pltpu.sync_copy(ones_v, deg_sp.at[cidx], add=True)
        return 0
    lax.fori_loop(0, DEG_FULL, win, 0)
    off = pl.multiple_of(base + DEG_FULL * W_E, 8)
    pltpu.sync_copy(col_hbm.at[pl.ds(off, DEG_TAIL)], cidx_t)
    pltpu.sync_copy(ones_t, deg_sp.at[cidx_t], add=True)

    plsc.subcore_barrier()

    @pl.when((s == 0) & (c == 0))
    def _():
        pltpu.sync_copy(deg_sp, deg0_hbm)

    @pl.when((s == 0) & (c == 1))
    def _():
        pltpu.sync_copy(deg_sp, deg1_hbm)


_deg_call = pl.kernel(
    _deg_body,
    out_type=(
        jax.ShapeDtypeStruct((NP,), jnp.float32),
        jax.ShapeDtypeStruct((NP,), jnp.float32),
    ),
    mesh=_mesh,
    scratch_types=[
        pltpu.VMEM((640,), jnp.float32),
        pltpu.VMEM((W_E,), jnp.float32),
        pltpu.VMEM((W_E,), jnp.int32),
        pltpu.VMEM((DEG_TAIL,), jnp.float32),
        pltpu.VMEM((DEG_TAIL,), jnp.int32),
        pltpu.VMEM_SHARED((NP,), jnp.float32),
    ],
)


def _copy_rows(src, dst, s):
    r0 = pl.multiple_of(s * ROWS_A, 8)

    @pl.when(s < 15)
    def _():
        pltpu.sync_copy(src.at[pl.ds(r0, ROWS_A)], dst.at[pl.ds(r0, ROWS_A)])

    @pl.when(s == 15)
    def _():
        pltpu.sync_copy(src.at[pl.ds(15 * ROWS_A, ROWS_B)],
                        dst.at[pl.ds(15 * ROWS_A, ROWS_B)])


def _agg_body(u0, u1, row_hbm, col_hbm, agg0, agg1,
              ridx, cidx, rows, agg_sp, gsem, ssem):
    c = lax.axis_index("c")
    s = lax.axis_index("s")

    @pl.when(c == 0)
    def _():
        _copy_rows(u0, agg_sp, s)

    @pl.when(c == 1)
    def _():
        _copy_rows(u1, agg_sp, s)

    plsc.subcore_barrier()

    def edge_loop(u_ref):
        def gather(j, b):
            pltpu.async_copy(u_ref.at[ridx.at[j]], rows.at[b], gsem)

        def chunk(k, _):
            cb = pl.multiple_of(s * SROWS + k * C_A, 8)
            pltpu.sync_copy(row_hbm.at[pl.ds(cb, C_A)], ridx)
            pltpu.sync_copy(col_hbm.at[pl.ds(cb, C_A)], cidx)
            gather(0, 0)

            def win(j, _):
                b = lax.rem(j, 2)
                pltpu.make_async_copy(u_ref.at[ridx.at[j]], rows.at[b],
                                      gsem).wait()
                pltpu.async_copy(rows.at[b], agg_sp.at[cidx.at[j]], ssem,
                                 add=True)

                @pl.when(j + 1 < C_A)
                def _():
                    @pl.when(j >= 1)
                    def _():
                        pltpu.make_async_copy(
                            rows.at[1 - b], agg_sp.at[cidx.at[j - 1]],
                            ssem).wait()
                    gather(j + 1, 1 - b)
                return 0
            lax.fori_loop(0, C_A, win, 0)
            pltpu.make_async_copy(rows.at[(C_A - 2) % 2],
                                  agg_sp.at[cidx.at[C_A - 2]], ssem).wait()
            pltpu.make_async_copy(rows.at[(C_A - 1) % 2],
                                  agg_sp.at[cidx.at[C_A - 1]], ssem).wait()
            return 0
        lax.fori_loop(0, NCHUNK, chunk, 0)

    @pl.when(c == 0)
    def _():
        edge_loop(u0)

    @pl.when(c == 1)
    def _():
        edge_loop(u1)

    plsc.subcore_barrier()

    @pl.when(c == 0)
    def _():
        _copy_rows(agg_sp, agg0, s)

    @pl.when(c == 1)
    def _():
        _copy_rows(agg_sp, agg1, s)


_agg_call = pl.kernel(
    _agg_body,
    out_type=(
        jax.ShapeDtypeStruct((N, HH), jnp.float32),
        jax.ShapeDtypeStruct((N, HH), jnp.float32),
    ),
    mesh=_mesh,
    scratch_types=[
        pltpu.VMEM((C_A, W_A), jnp.int32),
        pltpu.VMEM((C_A, W_A), jnp.int32),
        pltpu.VMEM((2, W_A, HH), jnp.float32),
        pltpu.VMEM_SHARED((N, HH), jnp.float32),
        pltpu.SemaphoreType.DMA,
        pltpu.SemaphoreType.DMA,
    ],
)


def _prep_body(deg0_ref, deg1_ref, dinv_ref):
    d = deg0_ref[...] + deg1_ref[...] + 1.0
    r = lax.rsqrt(d)
    r = r * (1.5 - 0.5 * d * r * r)
    dinv_ref[...] = r[:, None]


_prep_call = pl.pallas_call(
    _prep_body,
    out_shape=jax.ShapeDtypeStruct((NP, 1), jnp.float32),
    in_specs=[
        pl.BlockSpec((NP,), lambda: (0,)),
        pl.BlockSpec((NP,), lambda: (0,)),
    ],
    out_specs=pl.BlockSpec((NP, 1), lambda: (0, 0)),
)


def _mm1_body(x_ref, w_ref, dinv_ref, u0_ref, u1_ref):
    h = jnp.dot(x_ref[...], w_ref[...], preferred_element_type=jnp.float32,
                precision=lax.Precision.DEFAULT)
    u = h * dinv_ref[...]
    u0_ref[...] = u[:, :HH]
    u1_ref[...] = u[:, HH:]


_mm1_call = pl.pallas_call(
    _mm1_body,
    grid=(NGRID,),
    out_shape=(
        jax.ShapeDtypeStruct((N, HH), jnp.float32),
        jax.ShapeDtypeStruct((N, HH), jnp.float32),
    ),
    in_specs=[
        pl.BlockSpec((NB, F_IN), lambda i: (i, 0)),
        pl.BlockSpec((F_IN, H), lambda i: (0, 0)),
        pl.BlockSpec((NB, 1), lambda i: (i, 0)),
    ],
    out_specs=(
        pl.BlockSpec((NB, HH), lambda i: (i, 0)),
        pl.BlockSpec((NB, HH), lambda i: (i, 0)),
    ),
)


def _mid_body(a0_ref, a1_ref, dinv_ref, vs_ref, vo_ref, w_ref,
              n0_ref, n1_ref):
    a = jnp.concatenate([a0_ref[...], a1_ref[...]], axis=1)
    h = jnp.maximum(a * dinv_ref[...] * vs_ref[...] + vo_ref[...], 0.0)
    nu = jnp.dot(h, w_ref[...], preferred_element_type=jnp.float32,
                precision=lax.Precision.DEFAULT) * dinv_ref[...]
    n0_ref[...] = nu[:, :HH]
    n1_ref[...] = nu[:, HH:]


_mid_call = pl.pallas_call(
    _mid_body,
    grid=(NGRID,),
    out_shape=(
        jax.ShapeDtypeStruct((N, HH), jnp.float32),
        jax.ShapeDtypeStruct((N, HH), jnp.float32),
    ),
    in_specs=[
        pl.BlockSpec((NB, HH), lambda i: (i, 0)),
        pl.BlockSpec((NB, HH), lambda i: (i, 0)),
        pl.BlockSpec((NB, 1), lambda i: (i, 0)),
        pl.BlockSpec((1, H), lambda i: (0, 0)),
        pl.BlockSpec((1, H), lambda i: (0, 0)),
        pl.BlockSpec((H, H), lambda i: (0, 0)),
    ],
    out_specs=(
        pl.BlockSpec((NB, HH), lambda i: (i, 0)),
        pl.BlockSpec((NB, HH), lambda i: (i, 0)),
    ),
)


def _pool_body(a0_ref, a1_ref, dinv_ref, vs_ref, vo_ref,
               batch_ref, mw1_ref, mb1_ref, mw2_ref, mb2_ref,
               out_ref, s_acc, c_acc):
    i = pl.program_id(0)

    @pl.when(i == 0)
    def _():
        s_acc[...] = jnp.zeros_like(s_acc)
        c_acc[...] = jnp.zeros_like(c_acc)

    a = jnp.concatenate([a0_ref[...], a1_ref[...]], axis=1)
    h = jnp.maximum(a * dinv_ref[...] * vs_ref[...] + vo_ref[...], 0.0)
    b = batch_ref[...][:, 0]
    p = (b[None, :] == lax.broadcasted_iota(jnp.int32, (G, NB), 0)
         ).astype(jnp.float32)
    s_acc[...] += jnp.dot(p, h, preferred_element_type=jnp.float32,
                precision=lax.Precision.HIGHEST)
    c_acc[...] += jnp.sum(p, axis=1, keepdims=True)

    @pl.when(i == NGRID - 1)
    def _():
        pooled = s_acc[...] / jnp.maximum(c_acc[...], 1.0)
        z = jnp.maximum(
            jnp.dot(pooled, mw1_ref[...], preferred_element_type=jnp.float32,
                precision=lax.Precision.DEFAULT)
            + mb1_ref[...], 0.0)
        out_ref[...] = (
            jnp.dot(z, mw2_ref[...], preferred_element_type=jnp.float32,
                precision=lax.Precision.DEFAULT)
            + mb2_ref[...])


_pool_call = pl.pallas_call(
    _pool_body,
    grid=(NGRID,),
    out_shape=jax.ShapeDtypeStruct((G, 1), jnp.float32),
    in_specs=[
        pl.BlockSpec((NB, HH), lambda i: (i, 0)),
        pl.BlockSpec((NB, HH), lambda i: (i, 0)),
        pl.BlockSpec((NB, 1), lambda i: (i, 0)),
        pl.BlockSpec((1, H), lambda i: (0, 0)),
        pl.BlockSpec((1, H), lambda i: (0, 0)),
        pl.BlockSpec((NB, 1), lambda i: (i, 0)),
        pl.BlockSpec((H, H), lambda i: (0, 0)),
        pl.BlockSpec((1, H), lambda i: (0, 0)),
        pl.BlockSpec((H, 1), lambda i: (0, 0)),
        pl.BlockSpec((1, 1), lambda i: (0, 0)),
    ],
    out_specs=pl.BlockSpec((G, 1), lambda i: (0, 0)),
    scratch_shapes=[
        pltpu.VMEM((G, H), jnp.float32),
        pltpu.VMEM((G, 1), jnp.float32),
    ],
)


def kernel(x, edge_index, batch, W1, b1, g1, be1, rm1, rv1,
           W2, b2, g2, be2, rm2, rv2, W3, b3, g3, be3, rm3, rv3,
           mW1, mb1, mW2, mb2):
    row = edge_index[0].astype(jnp.int32)
    col = edge_index[1].astype(jnp.int32)
    row2 = row.reshape(EROWS, W_A)
    col2 = col.reshape(EROWS, W_A)
    batch2 = batch.astype(jnp.int32).reshape(N, 1)

    def fold(gv, bev, rmv, rvv, bv):
        sc = (gv * lax.rsqrt(rvv + EPS)).reshape(1, H)
        off = (bv * sc.reshape(H) + bev - rmv * sc.reshape(H)).reshape(1, H)
        return sc, off

    sc1, vo1 = fold(g1, be1, rm1, rv1, b1)
    sc2, vo2 = fold(g2, be2, rm2, rv2, b2)
    sc3, vo3 = fold(g3, be3, rm3, rv3, b3)

    deg0, deg1 = _deg_call(col)
    dinv = _prep_call(deg0, deg1)

    u0, u1 = _mm1_call(x, W1, dinv)
    a0, a1 = _agg_call(u0, u1, row2, col2)
    u0, u1 = _mid_call(a0, a1, dinv, sc1, vo1, W2)
    a0, a1 = _agg_call(u0, u1, row2, col2)
    u0, u1 = _mid_call(a0, a1, dinv, sc2, vo2, W3)
    a0, a1 = _agg_call(u0, u1, row2, col2)
    logit = _pool_call(a0, a1, dinv, sc3, vo3, batch2,
                       mW1, mb1.reshape(1, H), mW2, mb2.reshape(1, 1))
    return logit.reshape(G)
